# baseline (speedup 1.0000x reference)
import jax
import jax.numpy as jnp
from jax import lax
from jax.experimental import pallas as pl
from jax.experimental.pallas import tpu as pltpu
from jax.experimental.pallas import tpu_sc as plsc

VOCAB = 100000
OOV = 100
EV = VOCAB + OOV
B = 1024
L = 200
LPAD = 256
NCH = LPAD // 128
RB = 8
NC = 2
NS = 16
NW = NC * NS
RPW = B // NW


def _dense_body(pg_ref, vd_ref, out_ref):
    pg = pg_ref[...]
    out_ref[:, :VOCAB] = vd_ref[...] * pg
    out_ref[:, VOCAB:] = jnp.zeros((RB, OOV), jnp.float32)


def _vals_body(pg_ref, at_ref, out_ref):
    pg = pg_ref[...]
    out_ref[:, :L] = (1.0 - pg) * at_ref[...]
    out_ref[:, L:] = jnp.zeros((B, LPAD - L), jnp.float32)


def _sc_scatter_body(ids_hbm, vals_hbm, out_ref,
                     ids_v, vals_v, gidx_v, tot_v, dense_v, acc_v):
    c = lax.axis_index("c")
    s = lax.axis_index("s")
    wid = s * NC + c
    base = wid * RPW
    zeros16 = jnp.zeros((16,), jnp.float32)
    iota16 = lax.iota(jnp.int32, 16)
    lane_masks = [iota16 == l for l in range(16)]

    def row_body(r, carry):
        pltpu.sync_copy(ids_hbm.at[base + r], ids_v)
        pltpu.sync_copy(vals_hbm.at[base + r], vals_v)
        for j in range(NCH):
            for k in range(8):
                idx16 = ids_v[j, pl.ds(k * 16, 16)]
                plsc.store_scatter(acc_v, [idx16], zeros16)
        for j in range(NCH):
            for k in range(8):
                sl = pl.ds(k * 16, 16)
                idx16 = ids_v[j, sl]
                val16 = vals_v[j, sl]
                for l in range(16):
                    plsc.addupdate_scatter(acc_v, [idx16], val16,
                                           mask=lane_masks[l])
        off = (base + r) * EV
        for j in range(NCH):
            for k in range(8):
                sl = pl.ds(k * 16, 16)
                idx16 = ids_v[j, sl]
                tot_v[NCH * r + j, sl] = plsc.load_gather(acc_v, [idx16])
                gidx_v[NCH * r + j, sl] = idx16 + off
        return carry

    lax.fori_loop(0, RPW, row_body, 0)

    def rowb_body(r, carry):
        for j in range(NCH):
            pltpu.sync_copy(out_ref.at[gidx_v.at[NCH * r + j]], dense_v.at[j])
        for j in range(NCH):
            for k in range(8):
                sl = pl.ds(k * 16, 16)
                dense_v[j, sl] = dense_v[j, sl] + tot_v[NCH * r + j, sl]
        for j in range(NCH):
            pltpu.sync_copy(dense_v.at[j], out_ref.at[gidx_v.at[NCH * r + j]])
        return carry

    lax.fori_loop(0, RPW, rowb_body, 0)


def kernel(vocab_dists, attn_dists, p_gens, input_ids):
    vd = vocab_dists[0]
    at = attn_dists[0]
    pg = p_gens[0]

    dense = pl.pallas_call(
        _dense_body,
        grid=(B // RB,),
        in_specs=[
            pl.BlockSpec((RB, 1), lambda i: (i, 0)),
            pl.BlockSpec((RB, VOCAB), lambda i: (i, 0)),
        ],
        out_specs=pl.BlockSpec((RB, EV), lambda i: (i, 0)),
        out_shape=jax.ShapeDtypeStruct((B, EV), jnp.float32),
    )(pg, vd)

    vals = pl.pallas_call(
        _vals_body,
        in_specs=[
            pl.BlockSpec((B, 1), lambda: (0, 0)),
            pl.BlockSpec((B, L), lambda: (0, 0)),
        ],
        out_specs=pl.BlockSpec((B, LPAD), lambda: (0, 0)),
        out_shape=jax.ShapeDtypeStruct((B, LPAD), jnp.float32),
    )(pg, at)

    idpad = jnp.broadcast_to(input_ids[:, :1], (B, LPAD - L))
    ids3 = jnp.concatenate([input_ids, idpad], axis=1).reshape(B, NCH, 128)
    vals3 = vals.reshape(B, NCH, 128)

    sc_scatter = pl.kernel(
        _sc_scatter_body,
        out_type=(),
        mesh=plsc.VectorSubcoreMesh(core_axis_name="c", subcore_axis_name="s",
                                    num_cores=NC, num_subcores=NS),
        compiler_params=pltpu.CompilerParams(needs_layout_passes=False),
        scratch_types=[
            pltpu.VMEM((NCH, 128), jnp.int32),
            pltpu.VMEM((NCH, 128), jnp.float32),
            pltpu.VMEM((RPW * NCH, 128), jnp.int32),
            pltpu.VMEM((RPW * NCH, 128), jnp.float32),
            pltpu.VMEM((NCH, 128), jnp.float32),
            pltpu.VMEM((EV,), jnp.float32),
        ],
    )

    flat_ref = jax.new_ref(dense.reshape(B * EV))
    sc_scatter(ids3, vals3, flat_ref)
    return flat_ref[...].reshape(1, B, EV)


# speedup vs baseline: 3.9256x; 3.9256x over previous
import jax
import jax.numpy as jnp
from jax import lax
from jax.experimental import pallas as pl
from jax.experimental.pallas import tpu as pltpu
from jax.experimental.pallas import tpu_sc as plsc

VOCAB = 100000
OOV = 100
EV = VOCAB + OOV
B = 1024
L = 200
LPAD = 256
NCH = LPAD // 128
RB = 8
NT = 783
NC = 2
NS = 16
NW = NC * NS
RPW = B // NW
DELTA_WORDS = (B // 8) * NT * 8 * 128


def _vals_body(pg_ref, at_ref, out_ref):
    pg = pg_ref[...]
    out_ref[:, :L] = (1.0 - pg) * at_ref[...]
    out_ref[:, L:] = jnp.zeros((B, LPAD - L), jnp.float32)


def _sc_scatter_body(ids_hbm, vals_hbm, out_ref, ids_v, vals_v, gidx_v, tot_v,
                     acc_v):
    c = lax.axis_index("c")
    s = lax.axis_index("s")
    wid = s * NC + c
    base = wid * RPW
    zeros16 = jnp.zeros((16,), jnp.float32)
    iota16 = lax.iota(jnp.int32, 16)
    lane_masks = [iota16 == l for l in range(16)]

    def row_body(r, carry):
        b = base + r
        pltpu.sync_copy(ids_hbm.at[pl.ds(b * LPAD, LPAD)], ids_v)
        pltpu.sync_copy(vals_hbm.at[pl.ds(b * LPAD, LPAD)], vals_v)
        for g in range(16):
            sl = pl.ds(g * 16, 16)
            plsc.store_scatter(acc_v, [ids_v[sl]], zeros16)
        for g in range(16):
            sl = pl.ds(g * 16, 16)
            idx16 = ids_v[sl]
            val16 = vals_v[sl]
            for l in range(16):
                plsc.addupdate_scatter(acc_v, [idx16], val16,
                                       mask=lane_masks[l])
        rowoff = ((b >> 3) * NT) * 1024 + (b & 7) * 128
        for j in range(NCH):
            for k in range(8):
                sl16 = pl.ds((j * 8 + k) * 16, 16)
                idx16 = ids_v[sl16]
                tot_v[j, pl.ds(k * 16, 16)] = plsc.load_gather(acc_v, [idx16])
                gidx_v[j, pl.ds(k * 16, 16)] = (
                    (idx16 >> 7) * 1024 + (idx16 & 127) + rowoff)
        for j in range(NCH):
            pltpu.sync_copy(tot_v.at[j], out_ref.at[gidx_v.at[j]])
        return carry

    lax.fori_loop(0, RPW, row_body, 0)


def _merge_body(pg_ref, vd_ref, dl_ref, out_ref):
    pg = pg_ref[...]

    def body(jc, carry):
        for u in range(8):
            t = jc * 8 + u
            csl = pl.ds(t * 128, 128)
            out_ref[0, :, csl] = vd_ref[:, csl] * pg + dl_ref[0, t]
        return carry

    lax.fori_loop(0, 97, body, 0)
    for t in range(776, 781):
        csl = pl.ds(t * 128, 128)
        out_ref[0, :, csl] = vd_ref[:, csl] * pg + dl_ref[0, t]
    x = jnp.concatenate(
        [vd_ref[:, pl.ds(99968, 32)] * pg, jnp.zeros((RB, 96), jnp.float32)],
        axis=1)
    out_ref[0, :, pl.ds(99968, 128)] = x + dl_ref[0, 781]
    out_ref[0, :, pl.ds(100096, 4)] = dl_ref[0, 782, :, :4]


def kernel(vocab_dists, attn_dists, p_gens, input_ids):
    vd = vocab_dists[0]
    at = attn_dists[0]
    pg = p_gens[0]

    vals = pl.pallas_call(
        _vals_body,
        in_specs=[
            pl.BlockSpec((B, 1), lambda: (0, 0)),
            pl.BlockSpec((B, L), lambda: (0, 0)),
        ],
        out_specs=pl.BlockSpec((B, LPAD), lambda: (0, 0)),
        out_shape=jax.ShapeDtypeStruct((B, LPAD), jnp.float32),
    )(pg, at)

    idpad = jnp.broadcast_to(input_ids[:, :1], (B, LPAD - L))
    ids_flat = jnp.concatenate([input_ids, idpad], axis=1).reshape(B * LPAD)
    vals_flat = vals.reshape(B * LPAD)

    sc_scatter = pl.kernel(
        _sc_scatter_body,
        out_type=(),
        mesh=plsc.VectorSubcoreMesh(core_axis_name="c", subcore_axis_name="s",
                                    num_cores=NC, num_subcores=NS),
        compiler_params=pltpu.CompilerParams(needs_layout_passes=False),
        scratch_types=[
            pltpu.VMEM((LPAD,), jnp.int32),
            pltpu.VMEM((LPAD,), jnp.float32),
            pltpu.VMEM((NCH, 128), jnp.int32),
            pltpu.VMEM((NCH, 128), jnp.float32),
            pltpu.VMEM((EV,), jnp.float32),
        ],
    )

    delta_ref = jax.new_ref(jnp.zeros((DELTA_WORDS,), jnp.float32))
    sc_scatter(ids_flat, vals_flat, delta_ref)
    delta4 = delta_ref[...].reshape(B // 8, NT, 8, 128)

    out = pl.pallas_call(
        _merge_body,
        grid=(B // RB,),
        in_specs=[
            pl.BlockSpec((RB, 1), lambda i: (i, 0)),
            pl.BlockSpec((RB, VOCAB), lambda i: (i, 0)),
            pl.BlockSpec((1, NT, RB, 128), lambda i: (i, 0, 0, 0)),
        ],
        out_specs=pl.BlockSpec((1, RB, EV), lambda i: (0, i, 0)),
        out_shape=jax.ShapeDtypeStruct((1, B, EV), jnp.float32),
    )(pg, vd, delta4)
    return out


# speedup vs baseline: 4.9803x; 1.2687x over previous
import jax
import jax.numpy as jnp
from jax import lax
from jax.experimental import pallas as pl
from jax.experimental.pallas import tpu as pltpu
from jax.experimental.pallas import tpu_sc as plsc

VOCAB = 100000
OOV = 100
EV = VOCAB + OOV
B = 1024
L = 200
LPAD = 256
NCH = LPAD // 128
RB = 8
NT = 783
NC = 2
NS = 16
NW = NC * NS
RPW = B // NW
HRPW = RPW // 2
SPAN = (RPW // 8) * NT * 1024
CHUNK = 50112
NFILL = SPAN // CHUNK
DELTA_WORDS = NW * SPAN


def _vals_body(pg_ref, at_ref, out_ref):
    pg = pg_ref[...]
    out_ref[:, :L] = (1.0 - pg) * at_ref[...]
    out_ref[:, L:] = jnp.zeros((B, LPAD - L), jnp.float32)


def _sc_scatter_body(ids_hbm, vals_hbm, delta_hbm,
                     ids_a, vals_a, gidx_a, tot_a, acc_v, fill_sem, sc_sem):
    c = lax.axis_index("c")
    s = lax.axis_index("s")
    wid = s * NC + c
    base = wid * RPW
    sbase = wid * SPAN
    zeros16 = jnp.zeros((16,), jnp.float32)
    iota16 = lax.iota(jnp.int32, 16)
    lane_masks = [iota16 == l for l in range(16)]

    def zer_body(i, carry):
        acc_v[pl.ds(i * 16, 16)] = zeros16
        return carry

    lax.fori_loop(0, CHUNK // 16, zer_body, 0)
    fills = [
        pltpu.async_copy(acc_v.at[pl.ds(0, CHUNK)],
                         delta_hbm.at[pl.ds(sbase + i * CHUNK, CHUNK)],
                         fill_sem)
        for i in range(NFILL)
    ]
    for d in fills:
        d.wait()

    scatters = []
    for h in range(2):
        hbase = (base + h * HRPW) * LPAD
        pltpu.sync_copy(ids_hbm.at[pl.ds(hbase, HRPW * LPAD)], ids_a)
        pltpu.sync_copy(vals_hbm.at[pl.ds(hbase, HRPW * LPAD)], vals_a)

        def row_body(r, carry):
            b = base + h * HRPW + r
            rbase = r * LPAD
            for g in range(16):
                sl = pl.ds(rbase + g * 16, 16)
                plsc.store_scatter(acc_v, [ids_a[sl]], zeros16)
            for g in range(16):
                sl = pl.ds(rbase + g * 16, 16)
                idx16 = ids_a[sl]
                val16 = vals_a[sl]
                for l in range(16):
                    plsc.addupdate_scatter(acc_v, [idx16], val16,
                                           mask=lane_masks[l])
            rowoff = ((b >> 3) * NT) * 1024 + (b & 7) * 128
            q0 = (h * HRPW + r) * NCH
            for j in range(NCH):
                for k in range(8):
                    sl16 = pl.ds(rbase + (j * 8 + k) * 16, 16)
                    idx16 = ids_a[sl16]
                    tot_a[q0 + j, pl.ds(k * 16, 16)] = plsc.load_gather(
                        acc_v, [idx16])
                    gidx_a[q0 + j, pl.ds(k * 16, 16)] = (
                        (idx16 >> 7) * 1024 + (idx16 & 127) + rowoff)
            return carry

        lax.fori_loop(0, HRPW, row_body, 0)
        for q in range(h * HRPW * NCH, (h + 1) * HRPW * NCH):
            scatters.append(
                pltpu.async_copy(tot_a.at[q], delta_hbm.at[gidx_a.at[q]],
                                 sc_sem))
    for d in scatters:
        d.wait()


def _merge_body(pg_ref, vd_ref, dl_ref, out_ref):
    pg = pg_ref[...]
    for t in range(781):
        csl = pl.ds(t * 128, 128)
        out_ref[0, :, csl] = vd_ref[:, csl] * pg + dl_ref[0, t]
    x = jnp.concatenate(
        [vd_ref[:, pl.ds(99968, 32)] * pg, jnp.zeros((RB, 96), jnp.float32)],
        axis=1)
    out_ref[0, :, pl.ds(99968, 128)] = x + dl_ref[0, 781]
    out_ref[0, :, pl.ds(100096, 4)] = dl_ref[0, 782, :, :4]


def kernel(vocab_dists, attn_dists, p_gens, input_ids):
    vd = vocab_dists[0]
    at = attn_dists[0]
    pg = p_gens[0]

    vals = pl.pallas_call(
        _vals_body,
        in_specs=[
            pl.BlockSpec((B, 1), lambda: (0, 0)),
            pl.BlockSpec((B, L), lambda: (0, 0)),
        ],
        out_specs=pl.BlockSpec((B, LPAD), lambda: (0, 0)),
        out_shape=jax.ShapeDtypeStruct((B, LPAD), jnp.float32),
    )(pg, at)

    idpad = jnp.broadcast_to(input_ids[:, :1], (B, LPAD - L))
    ids_flat = jnp.concatenate([input_ids, idpad], axis=1).reshape(B * LPAD)
    vals_flat = vals.reshape(B * LPAD)

    sc_scatter = pl.kernel(
        _sc_scatter_body,
        out_type=jax.ShapeDtypeStruct((DELTA_WORDS,), jnp.float32),
        mesh=plsc.VectorSubcoreMesh(core_axis_name="c", subcore_axis_name="s",
                                    num_cores=NC, num_subcores=NS),
        compiler_params=pltpu.CompilerParams(needs_layout_passes=False),
        scratch_types=[
            pltpu.VMEM((HRPW * LPAD,), jnp.int32),
            pltpu.VMEM((HRPW * LPAD,), jnp.float32),
            pltpu.VMEM((RPW * NCH, 128), jnp.int32),
            pltpu.VMEM((RPW * NCH, 128), jnp.float32),
            pltpu.VMEM((EV,), jnp.float32),
            pltpu.SemaphoreType.DMA,
            pltpu.SemaphoreType.DMA,
        ],
    )

    delta = sc_scatter(ids_flat, vals_flat)
    delta4 = delta.reshape(B // 8, NT, 8, 128)

    out = pl.pallas_call(
        _merge_body,
        grid=(B // RB,),
        in_specs=[
            pl.BlockSpec((RB, 1), lambda i: (i, 0)),
            pl.BlockSpec((RB, VOCAB), lambda i: (i, 0)),
            pl.BlockSpec((1, NT, RB, 128), lambda i: (i, 0, 0, 0)),
        ],
        out_specs=pl.BlockSpec((1, RB, EV), lambda i: (0, i, 0)),
        out_shape=jax.ShapeDtypeStruct((1, B, EV), jnp.float32),
    )(pg, vd, delta4)
    return out


# speedup vs baseline: 4.9866x; 1.0013x over previous
import jax
import jax.numpy as jnp
from jax import lax
from jax.experimental import pallas as pl
from jax.experimental.pallas import tpu as pltpu
from jax.experimental.pallas import tpu_sc as plsc

VOCAB = 100000
OOV = 100
EV = VOCAB + OOV
B = 1024
L = 200
LPAD = 256
NCH = LPAD // 128
RB = 8
NT = 783
NC = 2
NS = 16
NW = NC * NS
RPW = B // NW
HRPW = RPW // 2
SPAN = (RPW // 8) * NT * 1024
CHUNK = 50112
NFILL = SPAN // CHUNK
DELTA_WORDS = NW * SPAN


def _vals_body(pg_ref, at_ref, out_ref):
    pg = pg_ref[...]
    out_ref[:, :L] = (1.0 - pg) * at_ref[...]
    out_ref[:, L:] = jnp.zeros((B, LPAD - L), jnp.float32)


def _sc_scatter_body(ids_hbm, vals_hbm, delta_hbm,
                     ids_a, vals_a, gidx_a, tot_a, acc_v, fill_sem, sc_sem):
    c = lax.axis_index("c")
    s = lax.axis_index("s")
    wid = s * NC + c
    base = wid * RPW
    sbase = wid * SPAN
    zeros16 = jnp.zeros((16,), jnp.float32)
    iota16 = lax.iota(jnp.int32, 16)
    lane_masks = [iota16 == l for l in range(16)]

    def zer_body(i, carry):
        acc_v[pl.ds(i * 16, 16)] = zeros16
        return carry

    lax.fori_loop(0, CHUNK // 16, zer_body, 0)
    fills = [
        pltpu.async_copy(acc_v.at[pl.ds(0, CHUNK)],
                         delta_hbm.at[pl.ds(sbase + i * CHUNK, CHUNK)],
                         fill_sem)
        for i in range(NFILL)
    ]
    for d in fills:
        d.wait()

    scatters = []
    for h in range(2):
        hbase = (base + h * HRPW) * LPAD
        pltpu.sync_copy(ids_hbm.at[pl.ds(hbase, HRPW * LPAD)], ids_a)
        pltpu.sync_copy(vals_hbm.at[pl.ds(hbase, HRPW * LPAD)], vals_a)

        def row_body(r, carry):
            b = base + h * HRPW + r
            rbase = r * LPAD
            for g in range(16):
                sl = pl.ds(rbase + g * 16, 16)
                plsc.store_scatter(acc_v, [ids_a[sl]], zeros16)
            for g in range(16):
                sl = pl.ds(rbase + g * 16, 16)
                idx16 = ids_a[sl]
                val16 = vals_a[sl]
                for l in range(16):
                    plsc.addupdate_scatter(acc_v, [idx16], val16,
                                           mask=lane_masks[l])
            rowoff = ((b >> 3) * NT) * 1024 + (b & 7) * 128
            q0 = (h * HRPW + r) * NCH
            for j in range(NCH):
                for k in range(8):
                    sl16 = pl.ds(rbase + (j * 8 + k) * 16, 16)
                    idx16 = ids_a[sl16]
                    tot_a[q0 + j, pl.ds(k * 16, 16)] = plsc.load_gather(
                        acc_v, [idx16])
                    gidx_a[q0 + j, pl.ds(k * 16, 16)] = (
                        (idx16 >> 7) * 1024 + (idx16 & 127) + rowoff)
            return carry

        lax.fori_loop(0, HRPW, row_body, 0)
        for q in range(h * HRPW * NCH, (h + 1) * HRPW * NCH):
            scatters.append(
                pltpu.async_copy(tot_a.at[q], delta_hbm.at[gidx_a.at[q]],
                                 sc_sem))
    for d in scatters:
        d.wait()


def _merge_body(pg_ref, vd_ref, dl_ref, out_ref):
    pg = pg_ref[...]
    for t in range(781):
        csl = pl.ds(t * 128, 128)
        out_ref[0, :, csl] = vd_ref[:, csl] * pg + dl_ref[pl.ds(t * 8, 8), :]
    x = jnp.concatenate(
        [vd_ref[:, pl.ds(99968, 32)] * pg, jnp.zeros((RB, 96), jnp.float32)],
        axis=1)
    out_ref[0, :, pl.ds(99968, 128)] = x + dl_ref[pl.ds(781 * 8, 8), :]
    out_ref[0, :, pl.ds(100096, 4)] = dl_ref[pl.ds(782 * 8, 8), :4]


def kernel(vocab_dists, attn_dists, p_gens, input_ids):
    vd = vocab_dists[0]
    at = attn_dists[0]
    pg = p_gens[0]

    vals = pl.pallas_call(
        _vals_body,
        in_specs=[
            pl.BlockSpec((B, 1), lambda: (0, 0)),
            pl.BlockSpec((B, L), lambda: (0, 0)),
        ],
        out_specs=pl.BlockSpec((B, LPAD), lambda: (0, 0)),
        out_shape=jax.ShapeDtypeStruct((B, LPAD), jnp.float32),
    )(pg, at)

    idpad = jnp.broadcast_to(input_ids[:, :1], (B, LPAD - L))
    ids_flat = jnp.concatenate([input_ids, idpad], axis=1).reshape(B * LPAD)
    vals_flat = vals.reshape(B * LPAD)

    sc_scatter = pl.kernel(
        _sc_scatter_body,
        out_type=jax.ShapeDtypeStruct((DELTA_WORDS,), jnp.float32),
        mesh=plsc.VectorSubcoreMesh(core_axis_name="c", subcore_axis_name="s",
                                    num_cores=NC, num_subcores=NS),
        compiler_params=pltpu.CompilerParams(needs_layout_passes=False),
        scratch_types=[
            pltpu.VMEM((HRPW * LPAD,), jnp.int32),
            pltpu.VMEM((HRPW * LPAD,), jnp.float32),
            pltpu.VMEM((RPW * NCH, 128), jnp.int32),
            pltpu.VMEM((RPW * NCH, 128), jnp.float32),
            pltpu.VMEM((EV,), jnp.float32),
            pltpu.SemaphoreType.DMA,
            pltpu.SemaphoreType.DMA,
        ],
    )

    delta = sc_scatter(ids_flat, vals_flat)
    delta2 = delta.reshape(DELTA_WORDS // 128, 128)

    out = pl.pallas_call(
        _merge_body,
        grid=(B // RB,),
        in_specs=[
            pl.BlockSpec((RB, 1), lambda i: (i, 0)),
            pl.BlockSpec((RB, VOCAB), lambda i: (i, 0)),
            pl.BlockSpec((NT * 8, 128), lambda i: (i, 0)),
        ],
        out_specs=pl.BlockSpec((1, RB, EV), lambda i: (0, i, 0)),
        out_shape=jax.ShapeDtypeStruct((1, B, EV), jnp.float32),
    )(pg, vd, delta2)
    return out


# speedup vs baseline: 5.5179x; 1.1065x over previous
import jax
import jax.numpy as jnp
from jax import lax
from jax.experimental import pallas as pl
from jax.experimental.pallas import tpu as pltpu
from jax.experimental.pallas import tpu_sc as plsc

VOCAB = 100000
OOV = 100
EV = VOCAB + OOV
B = 1024
L = 200
LPAD = 256
NCH = LPAD // 128
NS = 16
RPW = B // NS
QR = 16
DELTA_WORDS = EV * B
SPAN = DELTA_WORDS // NS
CHUNK = 50112
NFILL = SPAN // CHUNK
TAIL = SPAN - NFILL * CHUNK
VBLK = 1024


def _vals_body(pg_ref, at_ref, out_ref):
    pg = pg_ref[...]
    out_ref[:, :L] = (1.0 - pg) * at_ref[...]
    out_ref[:, L:] = jnp.zeros((B, LPAD - L), jnp.float32)


def _sc_scatter_body(ids_hbm, vals_hbm, delta_hbm,
                     ids_a, vals_a, gidx_a, tot_a, acc_v, fill_sem, sc_sem):
    s = lax.axis_index("s")
    base = s * RPW
    sbase = s * SPAN
    zeros16 = jnp.zeros((16,), jnp.float32)
    iota16 = lax.iota(jnp.int32, 16)
    lane_masks = [iota16 == l for l in range(16)]

    def zer_body(i, carry):
        acc_v[pl.ds(i * 16, 16)] = zeros16
        return carry

    lax.fori_loop(0, CHUNK // 16, zer_body, 0)
    fills = [
        pltpu.async_copy(acc_v.at[pl.ds(0, CHUNK)],
                         delta_hbm.at[pl.ds(sbase + i * CHUNK, CHUNK)],
                         fill_sem)
        for i in range(NFILL)
    ]
    fills.append(
        pltpu.async_copy(acc_v.at[pl.ds(0, TAIL)],
                         delta_hbm.at[pl.ds(sbase + NFILL * CHUNK, TAIL)],
                         fill_sem))
    for d in fills:
        d.wait()
    plsc.subcore_barrier()

    scatters = [None, None]
    for h in range(RPW // QR):
        p = h % 2
        if scatters[p] is not None:
            for d in scatters[p]:
                d.wait()
        hbase = (base + h * QR) * LPAD
        pltpu.sync_copy(ids_hbm.at[pl.ds(hbase, QR * LPAD)], ids_a)
        pltpu.sync_copy(vals_hbm.at[pl.ds(hbase, QR * LPAD)], vals_a)

        def row_body(r, carry):
            b = base + h * QR + r
            rbase = r * LPAD
            for g in range(16):
                sl = pl.ds(rbase + g * 16, 16)
                plsc.store_scatter(acc_v, [ids_a[sl]], zeros16)
            for g in range(16):
                sl = pl.ds(rbase + g * 16, 16)
                idx16 = ids_a[sl]
                val16 = vals_a[sl]
                for l in range(16):
                    plsc.addupdate_scatter(acc_v, [idx16], val16,
                                           mask=lane_masks[l])
            for j in range(NCH):
                for k in range(8):
                    sl16 = pl.ds(rbase + (j * 8 + k) * 16, 16)
                    idx16 = ids_a[sl16]
                    q = (p * QR + r) * NCH + j
                    tot_a[q, pl.ds(k * 16, 16)] = plsc.load_gather(
                        acc_v, [idx16])
                    gidx_a[q, pl.ds(k * 16, 16)] = idx16 * 1024 + b
            return carry

        lax.fori_loop(0, QR, row_body, 0)
        scatters[p] = [
            pltpu.async_copy(tot_a.at[q], delta_hbm.at[gidx_a.at[q]], sc_sem)
            for q in range(p * QR * NCH, (p + 1) * QR * NCH)
        ]
    for ds_ in scatters:
        for d in ds_:
            d.wait()


def _merge_body(pg_ref, vt_ref, dl_ref, out_ref):
    pgv = pg_ref[...]
    vbase = pl.program_id(0) * VBLK
    for g in range(VBLK // 8):
        x = vt_ref[pl.ds(g * 8, 8), :]
        x3 = x.reshape(8, 8, 128)
        dl = dl_ref[pl.ds(g * 8, 8)]
        keep = vbase + g * 8 < VOCAB
        out_ref[pl.ds(g * 8, 8)] = jnp.where(keep, x3 * pgv + dl, dl)


def kernel(vocab_dists, attn_dists, p_gens, input_ids):
    at = attn_dists[0]
    pg = p_gens[0]
    vt = vocab_dists.reshape(B, VOCAB).transpose(1, 0)

    vals = pl.pallas_call(
        _vals_body,
        in_specs=[
            pl.BlockSpec((B, 1), lambda: (0, 0)),
            pl.BlockSpec((B, L), lambda: (0, 0)),
        ],
        out_specs=pl.BlockSpec((B, LPAD), lambda: (0, 0)),
        out_shape=jax.ShapeDtypeStruct((B, LPAD), jnp.float32),
    )(pg, at)

    idpad = jnp.broadcast_to(input_ids[:, :1], (B, LPAD - L))
    ids_flat = jnp.concatenate([input_ids, idpad], axis=1).reshape(B * LPAD)
    vals_flat = vals.reshape(B * LPAD)

    sc_scatter = pl.kernel(
        _sc_scatter_body,
        out_type=jax.ShapeDtypeStruct((DELTA_WORDS,), jnp.float32),
        mesh=plsc.VectorSubcoreMesh(core_axis_name="c", subcore_axis_name="s",
                                    num_cores=1, num_subcores=NS),
        compiler_params=pltpu.CompilerParams(needs_layout_passes=False),
        scratch_types=[
            pltpu.VMEM((QR * LPAD,), jnp.int32),
            pltpu.VMEM((QR * LPAD,), jnp.float32),
            pltpu.VMEM((2 * QR * NCH, 128), jnp.int32),
            pltpu.VMEM((2 * QR * NCH, 128), jnp.float32),
            pltpu.VMEM((EV,), jnp.float32),
            pltpu.SemaphoreType.DMA,
            pltpu.SemaphoreType.DMA,
        ],
    )

    delta = sc_scatter(ids_flat, vals_flat)
    delta3 = delta.reshape(EV, 8, 128)
    pg8 = pg.reshape(8, 128)

    grid = (EV + VBLK - 1) // VBLK
    out3 = pl.pallas_call(
        _merge_body,
        grid=(grid,),
        in_specs=[
            pl.BlockSpec((8, 128), lambda i: (0, 0)),
            pl.BlockSpec((VBLK, B), lambda i: (i, 0)),
            pl.BlockSpec((VBLK, 8, 128), lambda i: (i, 0, 0)),
        ],
        out_specs=pl.BlockSpec((VBLK, 8, 128), lambda i: (i, 0, 0)),
        out_shape=jax.ShapeDtypeStruct((EV, 8, 128), jnp.float32),
    )(pg8, vt, delta3)

    return out3.reshape(EV, B).transpose(1, 0).reshape(1, B, EV)


# speedup vs baseline: 7.9954x; 1.4490x over previous
import jax
import jax.numpy as jnp
from jax import lax
from jax.experimental import pallas as pl
from jax.experimental.pallas import tpu as pltpu
from jax.experimental.pallas import tpu_sc as plsc

VOCAB = 100000
OOV = 100
EV = VOCAB + OOV
B = 1024
L = 200
LPAD = 256
NCH = LPAD // 128
NS = 16
RPW = B // NS
QR = 16
DELTA_WORDS = EV * B
SPAN = DELTA_WORDS // NS
CHUNK = 50112
NFILL = SPAN // CHUNK
TAIL = SPAN - NFILL * CHUNK
VBLK = 1024


def _vals_body(pg_ref, at_ref, out_ref):
    pg = pg_ref[...]
    out_ref[:, :L] = (1.0 - pg) * at_ref[...]
    out_ref[:, L:] = jnp.zeros((B, LPAD - L), jnp.float32)


def _sc_scatter_body(ids_hbm, vals_hbm, delta_hbm,
                     ids_a, vals_a, gidx_a, tot_a, acc_v, fill_sem, sc_sem):
    s = lax.axis_index("s")
    base = s * RPW
    sbase = s * SPAN
    zeros16 = jnp.zeros((16,), jnp.float32)
    iota16 = lax.iota(jnp.int32, 16)
    lane_masks = [iota16 == l for l in range(16)]

    def zer_body(i, carry):
        acc_v[pl.ds(i * 16, 16)] = zeros16
        return carry

    lax.fori_loop(0, CHUNK // 16, zer_body, 0)
    fills = [
        pltpu.async_copy(acc_v.at[pl.ds(0, CHUNK)],
                         delta_hbm.at[pl.ds(sbase + i * CHUNK, CHUNK)],
                         fill_sem)
        for i in range(NFILL)
    ]
    fills.append(
        pltpu.async_copy(acc_v.at[pl.ds(0, TAIL)],
                         delta_hbm.at[pl.ds(sbase + NFILL * CHUNK, TAIL)],
                         fill_sem))
    for d in fills:
        d.wait()
    plsc.subcore_barrier()

    scatters = [None, None]
    for h in range(RPW // QR):
        p = h % 2
        if scatters[p] is not None:
            for d in scatters[p]:
                d.wait()
        hbase = (base + h * QR) * LPAD
        pltpu.sync_copy(ids_hbm.at[pl.ds(hbase, QR * LPAD)], ids_a)
        pltpu.sync_copy(vals_hbm.at[pl.ds(hbase, QR * LPAD)], vals_a)

        def row_body(r, carry):
            b = base + h * QR + r
            rbase = r * LPAD
            for g in range(16):
                sl = pl.ds(rbase + g * 16, 16)
                plsc.store_scatter(acc_v, [ids_a[sl]], zeros16)
            for g in range(16):
                sl = pl.ds(rbase + g * 16, 16)
                idx16 = ids_a[sl]
                val16 = vals_a[sl]
                for l in range(16):
                    plsc.addupdate_scatter(acc_v, [idx16], val16,
                                           mask=lane_masks[l])
            for j in range(NCH):
                for k in range(8):
                    sl16 = pl.ds(rbase + (j * 8 + k) * 16, 16)
                    idx16 = ids_a[sl16]
                    q = (p * QR + r) * NCH + j
                    tot_a[q, pl.ds(k * 16, 16)] = plsc.load_gather(
                        acc_v, [idx16])
                    gidx_a[q, pl.ds(k * 16, 16)] = idx16 * 1024 + b
            return carry

        lax.fori_loop(0, QR, row_body, 0)
        scatters[p] = [
            pltpu.async_copy(tot_a.at[q], delta_hbm.at[gidx_a.at[q]], sc_sem)
            for q in range(p * QR * NCH, (p + 1) * QR * NCH)
        ]
    for ds_ in scatters:
        for d in ds_:
            d.wait()


def _merge_body(pg_ref, vt_ref, dl_ref, out_ref):
    pgv = pg_ref[...]
    vbase = pl.program_id(0) * VBLK
    dl = dl_ref[...].reshape(VBLK, B)
    rows = vbase + lax.broadcasted_iota(jnp.int32, (VBLK, 1), 0)
    out_ref[...] = jnp.where(rows < VOCAB, vt_ref[...] * pgv + dl, dl)


def kernel(vocab_dists, attn_dists, p_gens, input_ids):
    at = attn_dists[0]
    pg = p_gens[0]
    vt = vocab_dists.reshape(B, VOCAB).transpose(1, 0)

    vals = pl.pallas_call(
        _vals_body,
        in_specs=[
            pl.BlockSpec((B, 1), lambda: (0, 0)),
            pl.BlockSpec((B, L), lambda: (0, 0)),
        ],
        out_specs=pl.BlockSpec((B, LPAD), lambda: (0, 0)),
        out_shape=jax.ShapeDtypeStruct((B, LPAD), jnp.float32),
    )(pg, at)

    idpad = jnp.broadcast_to(input_ids[:, :1], (B, LPAD - L))
    ids_flat = jnp.concatenate([input_ids, idpad], axis=1).reshape(B * LPAD)
    vals_flat = vals.reshape(B * LPAD)

    sc_scatter = pl.kernel(
        _sc_scatter_body,
        out_type=jax.ShapeDtypeStruct((DELTA_WORDS,), jnp.float32),
        mesh=plsc.VectorSubcoreMesh(core_axis_name="c", subcore_axis_name="s",
                                    num_cores=1, num_subcores=NS),
        compiler_params=pltpu.CompilerParams(needs_layout_passes=False),
        scratch_types=[
            pltpu.VMEM((QR * LPAD,), jnp.int32),
            pltpu.VMEM((QR * LPAD,), jnp.float32),
            pltpu.VMEM((2 * QR * NCH, 128), jnp.int32),
            pltpu.VMEM((2 * QR * NCH, 128), jnp.float32),
            pltpu.VMEM((EV,), jnp.float32),
            pltpu.SemaphoreType.DMA,
            pltpu.SemaphoreType.DMA,
        ],
    )

    delta = sc_scatter(ids_flat, vals_flat)
    delta3 = delta.reshape(EV, 8, 128)
    pg1 = pg.reshape(1, B)

    grid = (EV + VBLK - 1) // VBLK
    out2 = pl.pallas_call(
        _merge_body,
        grid=(grid,),
        in_specs=[
            pl.BlockSpec((1, B), lambda i: (0, 0)),
            pl.BlockSpec((VBLK, B), lambda i: (i, 0)),
            pl.BlockSpec((VBLK, 8, 128), lambda i: (i, 0, 0)),
        ],
        out_specs=pl.BlockSpec((VBLK, B), lambda i: (i, 0)),
        out_shape=jax.ShapeDtypeStruct((EV, B), jnp.float32),
    )(pg1, vt, delta3)

    return out2.transpose(1, 0).reshape(1, B, EV)
